# baseline (speedup 1.0000x reference)
import functools

import numpy as np
import jax
import jax.numpy as jnp
from jax import lax
from jax.experimental import pallas as pl
from jax.experimental.pallas import tpu as pltpu
from jax.experimental.pallas import tpu_sc as plsc


def _positional_table(T, E):
    pos = np.arange(T, dtype=np.float32)[:, None]
    i = np.arange(E, dtype=np.float32)[None, :]
    angles = pos / np.power(10000.0, 2.0 * i / E)
    table = np.array(angles, dtype=np.float32)
    table[:, 0::2] = np.sin(table[:, 0::2])
    table[:, 1::2] = np.cos(table[:, 1::2])
    return table


def kernel(x):
    B, T, E = x.shape
    TE = T * E
    tab = _positional_table(T, E).reshape(TE)
    tab_splat = jnp.asarray(np.broadcast_to(tab[:, None], (TE, 128)).copy())
    xt = x.reshape(B, TE).T

    info = plsc.get_sparse_core_info()
    NW = info.num_cores * info.num_subcores
    rows_per_w = TE // NW
    GR, GC = 8, 2048
    col_halves = B // GC
    n_chunks = (rows_per_w // GR) * col_halves
    n_pairs = n_chunks // 2

    mesh = plsc.VectorSubcoreMesh(core_axis_name="c", subcore_axis_name="s")

    @functools.partial(
        pl.kernel,
        out_type=jax.ShapeDtypeStruct((TE, B), jnp.float32),
        mesh=mesh,
        scratch_types=[
            pltpu.VMEM((GR, GC), jnp.float32),
            pltpu.VMEM((GR, GC), jnp.float32),
            pltpu.VMEM((GR, GC), jnp.float32),
            pltpu.VMEM((GR, GC), jnp.float32),
            pltpu.VMEM((rows_per_w, 128), jnp.float32),
            pltpu.SemaphoreType.DMA,
            pltpu.SemaphoreType.DMA,
            pltpu.SemaphoreType.DMA,
            pltpu.SemaphoreType.DMA,
        ],
    )
    def sc_add(x_hbm, tab_hbm, out_hbm, in0, in1, out0, out1, tabs,
               si0, si1, so0, so1):
        cid = lax.axis_index("c")
        sid = lax.axis_index("s")
        wid = sid * info.num_cores + cid
        row0 = wid * rows_per_w
        ins, outs, sis, sos = (in0, in1), (out0, out1), (si0, si1), (so0, so1)
        pltpu.sync_copy(tab_hbm.at[pl.ds(row0, rows_per_w)], tabs)

        def src(i):
            band = i // col_halves
            colh = i % col_halves
            return (pl.ds(row0 + band * GR, GR), pl.ds(colh * GC, GC))

        for b in range(2):
            r, c = src(b)
            pltpu.async_copy(x_hbm.at[r, c], ins[b], sis[b])

        def pair(p, carry):
            for b in range(2):
                i = 2 * p + b
                r, c = src(i)
                pltpu.make_async_copy(x_hbm.at[r, c], ins[b], sis[b]).wait()

                @pl.when(p > 0)
                def _():
                    r2, c2 = src(i - 2)
                    pltpu.make_async_copy(
                        outs[b], out_hbm.at[r2, c2], sos[b]).wait()

                lrow = (i // col_halves) * GR
                for rr in range(GR):
                    a = tabs[lrow + rr, pl.ds(0, 16)]

                    @plsc.parallel_loop(0, GC, step=16, unroll=4)
                    def _(o):
                        outs[b][rr, pl.ds(o, 16)] = (
                            ins[b][rr, pl.ds(o, 16)] + a)

                @pl.when(i + 2 < n_chunks)
                def _():
                    r3, c3 = src(i + 2)
                    pltpu.async_copy(x_hbm.at[r3, c3], ins[b], sis[b])

                pltpu.async_copy(outs[b], out_hbm.at[r, c], sos[b])
            return carry

        lax.fori_loop(0, n_pairs, pair, 0)

        for b in range(2):
            r, c = src(n_chunks - 2 + b)
            pltpu.make_async_copy(outs[b], out_hbm.at[r, c], sos[b]).wait()

    out = sc_add(xt, tab_splat)
    return out.T.reshape(B, T, E)
